# baseline (speedup 1.0000x reference)
import jax
import jax.numpy as jnp
from jax import lax
from jax.experimental import pallas as pl
from jax.experimental.pallas import tpu as pltpu
from jax.experimental.pallas import tpu_sc as plsc

_NW = 32
_CR = 8
_NBUF = 2


def _make_sc_kernel(B, S, D):
    tr = S // _NW
    nch = tr // _CR
    groups = nch // _NBUF

    def body(x_hbm, t_hbm, o_hbm, xbuf, tbuf, *sems):
        xsems = sems[:_NBUF]
        tsems = sems[_NBUF:2 * _NBUF]
        osems = sems[2 * _NBUF:]
        c = lax.axis_index("c")
        s = lax.axis_index("s")
        wid = s * 2 + c
        tb0 = wid * tr

        def tcopy(i, b):
            return pltpu.make_async_copy(
                t_hbm.at[pl.ds(tb0 + i * _CR, _CR)], tbuf.at[b], tsems[b])

        def xcopy(i, b):
            return pltpu.make_async_copy(
                x_hbm.at[:, pl.ds(tb0 + i * _CR, _CR), :],
                xbuf.at[b], xsems[b])

        def ocopy(i, b):
            return pltpu.make_async_copy(
                xbuf.at[b],
                o_hbm.at[:, pl.ds(tb0 + i * _CR, _CR), :], osems[b])

        for b in range(_NBUF):
            tcopy(b, b).start()
            xcopy(b, b).start()

        def group(g, carry):
            for b in range(_NBUF):
                i = g * _NBUF + b
                pl.when(i >= _NBUF)(lambda: ocopy(i - _NBUF, b).wait())
                tcopy(i, b).wait()
                xcopy(i, b).wait()

                def row(rr, carry2):
                    for bb in range(B):
                        for j in range(D // 16):
                            sl = pl.ds(j * 16, 16)
                            plsc.addupdate(xbuf.at[b, bb, rr, sl],
                                           tbuf[b, rr, sl])
                    return carry2

                lax.fori_loop(0, _CR, row, 0, unroll=False)
                ocopy(i, b).start()

                def prefetch():
                    tcopy(i + _NBUF, b).start()
                    xcopy(i + _NBUF, b).start()

                pl.when(i + _NBUF < nch)(prefetch)
            return carry

        lax.fori_loop(0, groups, group, 0, unroll=False)
        for b in range(_NBUF):
            ocopy(nch - _NBUF + b, b).wait()

    return pl.kernel(
        body,
        out_type=jax.ShapeDtypeStruct((B, S, D), jnp.float32),
        mesh=plsc.VectorSubcoreMesh(core_axis_name="c", subcore_axis_name="s"),
        scratch_types=[
            pltpu.VMEM((_NBUF, B, _CR, D), jnp.float32),
            pltpu.VMEM((_NBUF, _CR, D), jnp.float32),
        ] + [pltpu.SemaphoreType.DMA] * (3 * _NBUF),
    )


def kernel(x, embed_table):
    B, S, D = x.shape
    return _make_sc_kernel(B, S, D)(x, embed_table)


# speedup vs baseline: 1.8019x; 1.8019x over previous
import jax
import jax.numpy as jnp
from jax import lax
from jax.experimental import pallas as pl
from jax.experimental.pallas import tpu as pltpu
from jax.experimental.pallas import tpu_sc as plsc

_NW = 32
_CR = 8
_NBUF = 2


def _make_sc_kernel(B, S, D):
    tr = S // _NW
    nch = tr // _CR
    groups = nch // _NBUF

    def body(x_hbm, t_hbm, o_hbm, xbuf, tbuf, *sems):
        xsems = sems[:_NBUF]
        tsems = sems[_NBUF:2 * _NBUF]
        osems = sems[2 * _NBUF:]
        c = lax.axis_index("c")
        s = lax.axis_index("s")
        wid = s * 2 + c
        tb0 = wid * tr

        def tcopy(i, b):
            return pltpu.make_async_copy(
                t_hbm.at[pl.ds(tb0 + i * _CR, _CR)], tbuf.at[b], tsems[b])

        def xcopy(i, b, bb):
            return pltpu.make_async_copy(
                x_hbm.at[pl.ds(bb * S + tb0 + i * _CR, _CR)],
                xbuf.at[b, bb], xsems[b])

        def ocopy(i, b, bb):
            return pltpu.make_async_copy(
                xbuf.at[b, bb],
                o_hbm.at[pl.ds(bb * S + tb0 + i * _CR, _CR)], osems[b])

        for b in range(_NBUF):
            tcopy(b, b).start()
            for bb in range(B):
                xcopy(b, b, bb).start()

        def group(g, carry):
            for b in range(_NBUF):
                i = g * _NBUF + b
                def drain():
                    for bb in range(B):
                        ocopy(i - _NBUF, b, bb).wait()
                pl.when(i >= _NBUF)(drain)
                tcopy(i, b).wait()
                for bb in range(B):
                    xcopy(i, b, bb).wait()

                    def row(r, carry2):
                        for j in range(D // 16):
                            sl = pl.ds(j * 16, 16)
                            plsc.addupdate(xbuf.at[b, bb, r, sl],
                                           tbuf[b, r, sl])
                        return carry2

                    lax.fori_loop(0, _CR, row, 0, unroll=False)
                    ocopy(i, b, bb).start()

                def prefetch():
                    tcopy(i + _NBUF, b).start()
                    for bb in range(B):
                        xcopy(i + _NBUF, b, bb).start()

                pl.when(i + _NBUF < nch)(prefetch)
            return carry

        lax.fori_loop(0, groups, group, 0, unroll=False)
        for b in range(_NBUF):
            for bb in range(B):
                ocopy(nch - _NBUF + b, b, bb).wait()

    return pl.kernel(
        body,
        out_type=jax.ShapeDtypeStruct((B * S, D), jnp.float32),
        mesh=plsc.VectorSubcoreMesh(core_axis_name="c", subcore_axis_name="s"),
        scratch_types=[
            pltpu.VMEM((_NBUF, B, _CR, D), jnp.float32),
            pltpu.VMEM((_NBUF, _CR, D), jnp.float32),
        ] + [pltpu.SemaphoreType.DMA] * (3 * _NBUF),
    )


def kernel(x, embed_table):
    B, S, D = x.shape
    x2 = x.reshape(B * S, D)
    out = _make_sc_kernel(B, S, D)(x2, embed_table)
    return out.reshape(B, S, D)


# speedup vs baseline: 1.8116x; 1.0054x over previous
import jax
import jax.numpy as jnp
from jax import lax
from jax.experimental import pallas as pl
from jax.experimental.pallas import tpu as pltpu
from jax.experimental.pallas import tpu_sc as plsc

_NW = 32
_CR = 8
_NBUF = 2


def _make_sc_kernel(B, S, D):
    tr = S // _NW
    nch = tr // _CR
    groups = nch // _NBUF

    nslot = 2 * B
    nitems = nch * B
    igroups = nitems // nslot

    def body(x_hbm, t_hbm, o_hbm, xbuf, tbuf, *sems):
        xsems = sems[:nslot]
        tsems = sems[nslot:nslot + _NBUF]
        osems = sems[nslot + _NBUF:]
        c = lax.axis_index("c")
        s = lax.axis_index("s")
        wid = s * 2 + c
        tb0 = wid * tr

        def tcopy(i, tb):
            return pltpu.make_async_copy(
                t_hbm.at[pl.ds(tb0 + i * _CR, _CR)], tbuf.at[tb], tsems[tb])

        def xcopy(i, bb, u):
            return pltpu.make_async_copy(
                x_hbm.at[pl.ds(bb * S + tb0 + i * _CR, _CR)],
                xbuf.at[u], xsems[u])

        def ocopy(i, bb, u):
            return pltpu.make_async_copy(
                xbuf.at[u],
                o_hbm.at[pl.ds(bb * S + tb0 + i * _CR, _CR)], osems[u])

        for tb in range(_NBUF):
            tcopy(tb, tb).start()
        for u in range(nslot):
            xcopy(u // B, u % B, u).start()

        def group(g, carry):
            for u in range(nslot):
                du, bb = u // B, u % B
                i = g * _NBUF + du
                tb = du
                if bb == 0:
                    tcopy(i, tb).wait()
                pl.when(g > 0)(lambda: ocopy(i - _NBUF, bb, u).wait())
                xcopy(i, bb, u).wait()

                def row(r, carry2):
                    for j in range(D // 16):
                        sl = pl.ds(j * 16, 16)
                        plsc.addupdate(xbuf.at[u, r, sl], tbuf[tb, r, sl])
                    return carry2

                lax.fori_loop(0, _CR, row, 0, unroll=False)
                ocopy(i, bb, u).start()
                pl.when(i + _NBUF < nch)(
                    lambda: xcopy(i + _NBUF, bb, u).start())
                if bb == B - 1:
                    pl.when(i + _NBUF < nch)(
                        lambda: tcopy(i + _NBUF, tb).start())
            return carry

        lax.fori_loop(0, igroups, group, 0, unroll=False)
        for u in range(nslot):
            ocopy(nch - _NBUF + u // B, u % B, u).wait()

    return pl.kernel(
        body,
        out_type=jax.ShapeDtypeStruct((B * S, D), jnp.float32),
        mesh=plsc.VectorSubcoreMesh(core_axis_name="c", subcore_axis_name="s"),
        scratch_types=[
            pltpu.VMEM((nslot, _CR, D), jnp.float32),
            pltpu.VMEM((_NBUF, _CR, D), jnp.float32),
        ] + [pltpu.SemaphoreType.DMA] * (2 * nslot + _NBUF),
    )


def kernel(x, embed_table):
    B, S, D = x.shape
    x2 = x.reshape(B * S, D)
    out = _make_sc_kernel(B, S, D)(x2, embed_table)
    return out.reshape(B, S, D)
